# Initial kernel scaffold; baseline (speedup 1.0000x reference)
import jax
import jax.numpy as jnp
from jax.experimental import pallas as pl


def kernel(x, edge_index, params):
    raise NotImplementedError("write your pallas kernel here")


# speedup vs baseline: 3.6980x; 3.6980x over previous
import functools

import jax
import jax.numpy as jnp
from jax import lax
from jax.experimental import pallas as pl
from jax.experimental.pallas import tpu as pltpu
from jax.experimental.pallas import tpu_sc as plsc

NN = 10000
NP = 10240
EE = 160000
DD = 256
HH = 512
NE = 3
NB = 1024
GRID = NP // NB

_mesh = plsc.VectorSubcoreMesh(core_axis_name="c", subcore_axis_name="s")


@functools.partial(
    pl.kernel,
    out_type=jax.ShapeDtypeStruct((2 * NP, 128), jnp.float32),
    mesh=_mesh,
    scratch_types=[
        pltpu.VMEM((40,), jnp.int32),
        pltpu.VMEM((40, 128), jnp.float32),
        pltpu.VMEM_SHARED((NP, 128), jnp.float32),
    ],
)
def _sc_deg(src_hbm, zeros_hbm, ones_hbm, out_hbm, idx_v, ones_v, acc):
    c = lax.axis_index("c")
    s = lax.axis_index("s")
    pltpu.sync_copy(zeros_hbm.at[pl.ds(s * 640, 640)],
                    acc.at[pl.ds(s * 640, 640)])
    pltpu.sync_copy(ones_hbm, ones_v)
    plsc.subcore_barrier()
    base = (c * 16 + s) * 5000

    def body(i, _):
        pltpu.sync_copy(src_hbm.at[pl.ds(base + i * 40, 40)], idx_v)
        pltpu.sync_copy(ones_v, acc.at[idx_v], add=True)
        return 0

    lax.fori_loop(0, 125, body, 0)
    plsc.subcore_barrier()
    pltpu.sync_copy(acc.at[pl.ds(s * 640, 640)],
                    out_hbm.at[pl.ds(c * NP + s * 640, 640)])


@functools.partial(
    pl.kernel,
    out_type=jax.ShapeDtypeStruct((2 * NP, 128), jnp.float32),
    mesh=_mesh,
    scratch_types=[
        pltpu.VMEM((80,), jnp.int32),
        pltpu.VMEM((80,), jnp.int32),
        pltpu.VMEM((80,), jnp.int32),
        pltpu.VMEM((80, 128), jnp.float32),
        pltpu.VMEM_SHARED((NP, 128), jnp.float32),
        pltpu.SemaphoreType.DMA,
    ],
)
def _sc_prop(v_hbm, src_hbm, dst_hbm, zeros_hbm, out_hbm,
             idx_s, idx_sa, idx_d, rows, acc, sem):
    c = lax.axis_index("c")
    s = lax.axis_index("s")
    pltpu.sync_copy(zeros_hbm.at[pl.ds(s * 640, 640)],
                    acc.at[pl.ds(s * 640, 640)])
    plsc.subcore_barrier()
    ebase = s * 10000
    off = c * NP

    def body(i, _):
        e0 = ebase + i * 80
        pltpu.sync_copy(src_hbm.at[pl.ds(e0, 80)], idx_s)
        pltpu.sync_copy(dst_hbm.at[pl.ds(e0, 80)], idx_d)
        for k in range(5):
            idx_sa[pl.ds(k * 16, 16)] = idx_s[pl.ds(k * 16, 16)] + off
        pltpu.async_copy(v_hbm.at[idx_sa], rows, sem).wait()
        pltpu.sync_copy(rows, acc.at[idx_d], add=True)
        return 0

    lax.fori_loop(0, 125, body, 0)
    plsc.subcore_barrier()
    pltpu.sync_copy(acc.at[pl.ds(s * 640, 640)],
                    out_hbm.at[pl.ds(c * NP + s * 640, 640)])


def _ln(x, g, b):
    mu = jnp.mean(x, axis=-1, keepdims=True)
    var = jnp.mean((x - mu) ** 2, axis=-1, keepdims=True)
    return (x - mu) / jnp.sqrt(var + 1e-5) * g + b


def _gelu(x):
    return 0.5 * x * (1.0 + lax.erf(x * 0.7071067811865476))


def _dinv(dp_ref):
    deg = dp_ref[0][:, :1] + dp_ref[1][:, :1]
    return jnp.where(deg > 0.0, lax.rsqrt(jnp.where(deg > 0.0, deg, 1.0)),
                     0.0)


def _tca_body(x_ref, dp_ref, wg1_ref, bg1_ref, g_ref, b_ref, wg2_ref,
              bg2_ref, xs_ref, ew_ref):
    dinv = _dinv(dp_ref)
    xs = x_ref[...] * dinv
    xs_ref[0] = xs[:, :128]
    xs_ref[1] = xs[:, 128:]
    w = jnp.dot(x_ref[...], wg1_ref[...],
                preferred_element_type=jnp.float32) + bg1_ref[...]
    w = _gelu(_ln(w, g_ref[...], b_ref[...]))
    logits = jnp.dot(w, wg2_ref[...],
                     preferred_element_type=jnp.float32) + bg2_ref[...]
    m = jnp.max(logits, axis=-1, keepdims=True)
    ex = jnp.exp(logits - m)
    ew_ref[...] = ex / jnp.sum(ex, axis=-1, keepdims=True)


def _tca(x_p, dp, wg1, bg1, g, b, wg2, bg2):
    full = lambda shp: pl.BlockSpec(shp, lambda i: (0,) * len(shp))
    return pl.pallas_call(
        _tca_body,
        grid=(GRID,),
        in_specs=[
            pl.BlockSpec((NB, DD), lambda i: (i, 0)),
            pl.BlockSpec((2, NB, 128), lambda i: (0, i, 0)),
            full((DD, 128)), full((1, 128)), full((1, 128)), full((1, 128)),
            full((128, 128)), full((1, 128)),
        ],
        out_specs=[
            pl.BlockSpec((2, NB, 128), lambda i: (0, i, 0)),
            pl.BlockSpec((NB, 128), lambda i: (i, 0)),
        ],
        out_shape=[
            jax.ShapeDtypeStruct((2, NP, 128), jnp.float32),
            jax.ShapeDtypeStruct((NP, 128), jnp.float32),
        ],
    )(x_p, dp, wg1, bg1, g, b, wg2, bg2)


def _tcb_body(y1_ref, dp_ref, t_ref):
    dinv = _dinv(dp_ref)
    sc = -(dinv * dinv)
    t_ref[0] = y1_ref[0] * sc
    t_ref[1] = y1_ref[1] * sc


def _tcb(y1, dp):
    return pl.pallas_call(
        _tcb_body,
        grid=(GRID,),
        in_specs=[
            pl.BlockSpec((2, NB, 128), lambda i: (0, i, 0)),
            pl.BlockSpec((2, NB, 128), lambda i: (0, i, 0)),
        ],
        out_specs=pl.BlockSpec((2, NB, 128), lambda i: (0, i, 0)),
        out_shape=jax.ShapeDtypeStruct((2, NP, 128), jnp.float32),
    )(y1, dp)


def _tcc_body(x_ref, y1_ref, y2_ref, dp_ref, w1_ref, b1_ref, g1_ref,
              be1_ref, h_ref, hs_ref):
    dinv = _dinv(dp_ref)
    x = x_ref[...]
    tx1 = -dinv * jnp.concatenate([y1_ref[0], y1_ref[1]], axis=1)
    tx2 = -2.0 * dinv * jnp.concatenate([y2_ref[0], y2_ref[1]], axis=1) - x
    cat = jnp.concatenate([x, tx1, tx2], axis=1)
    for e in range(NE):
        pre = jnp.dot(cat, w1_ref[e],
                      preferred_element_type=jnp.float32) + b1_ref[e]
        h = _gelu(_ln(pre, g1_ref[e], be1_ref[e]))
        h_ref[e] = h
        hs = h * dinv
        for j in range(4):
            ci = e * 4 + j
            hs_ref[ci // 2, ci % 2] = hs[:, j * 128:(j + 1) * 128]


def _tcc(x_p, y1, y2, dp, w1, b1, g1, be1):
    full = lambda shp: pl.BlockSpec(shp, lambda i: (0,) * len(shp))
    return pl.pallas_call(
        _tcc_body,
        grid=(GRID,),
        in_specs=[
            pl.BlockSpec((NB, DD), lambda i: (i, 0)),
            pl.BlockSpec((2, NB, 128), lambda i: (0, i, 0)),
            pl.BlockSpec((2, NB, 128), lambda i: (0, i, 0)),
            pl.BlockSpec((2, NB, 128), lambda i: (0, i, 0)),
            full((NE, 3 * DD, HH)), full((NE, 1, HH)),
            full((NE, 1, HH)), full((NE, 1, HH)),
        ],
        out_specs=[
            pl.BlockSpec((NE, NB, HH), lambda i: (0, i, 0)),
            pl.BlockSpec((6, 2, NB, 128), lambda i: (0, 0, i, 0)),
        ],
        out_shape=[
            jax.ShapeDtypeStruct((NE, NP, HH), jnp.float32),
            jax.ShapeDtypeStruct((6, 2, NP, 128), jnp.float32),
        ],
    )(x_p, y1, y2, dp, w1, b1, g1, be1)


def _tcd_body(h_ref, z0, z1, z2, z3, z4, z5, x_ref, dp_ref, ew_ref,
              w2_ref, b2_ref, g2_ref, be2_ref, wu1_ref, bu1_ref, gu_ref,
              bu_ref, wu2_ref, bu2_ref, ws_ref, un_ref):
    dinv = _dinv(dp_ref)
    zrefs = (z0, z1, z2, z3, z4, z5)
    ws = jnp.zeros((NB, DD), jnp.float32)
    ew = ew_ref[...]
    for e in range(NE):
        zcols = [zrefs[(4 * e + j) // 2][(4 * e + j) % 2] for j in range(4)]
        ph = -dinv * jnp.concatenate(zcols, axis=1)
        cat = jnp.concatenate([h_ref[e], ph], axis=1)
        pre = jnp.dot(cat, w2_ref[e],
                      preferred_element_type=jnp.float32) + b2_ref[e]
        o = _ln(pre, g2_ref[e], be2_ref[e])
        ws = ws + o * ew[:, e:e + 1]
    ws_ref[...] = ws
    ucat = jnp.concatenate([ws, x_ref[...]], axis=1)
    u = jnp.dot(ucat, wu1_ref[...],
                preferred_element_type=jnp.float32) + bu1_ref[...]
    lane = lax.broadcasted_iota(jnp.int32, (1, 128), 1)
    m = lane < 64
    mu = jnp.sum(jnp.where(m, u, 0.0), axis=-1, keepdims=True) / 64.0
    d = jnp.where(m, u - mu, 0.0)
    var = jnp.sum(d * d, axis=-1, keepdims=True) / 64.0
    un = (u - mu) / jnp.sqrt(var + 1e-5) * gu_ref[...] + bu_ref[...]
    un = jnp.where(m, _gelu(un), 0.0)
    u2 = jnp.dot(un, wu2_ref[...],
                 preferred_element_type=jnp.float32) + bu2_ref[...]
    un_ref[...] = 1.0 / (1.0 + jnp.exp(-u2))


def _tcd(h, zs, x_p, dp, ewp, w2, b2, g2, be2, wu1, bu1, gu, bu, wu2, bu2):
    full = lambda shp: pl.BlockSpec(shp, lambda i: (0,) * len(shp))
    zspec = pl.BlockSpec((2, NB, 128), lambda i: (0, i, 0))
    return pl.pallas_call(
        _tcd_body,
        grid=(GRID,),
        in_specs=[
            pl.BlockSpec((NE, NB, HH), lambda i: (0, i, 0)),
            zspec, zspec, zspec, zspec, zspec, zspec,
            pl.BlockSpec((NB, DD), lambda i: (i, 0)),
            pl.BlockSpec((2, NB, 128), lambda i: (0, i, 0)),
            pl.BlockSpec((NB, 128), lambda i: (i, 0)),
            full((NE, 2 * HH, DD)), full((NE, 1, DD)),
            full((NE, 1, DD)), full((NE, 1, DD)),
            full((2 * DD, 128)), full((1, 128)), full((1, 128)),
            full((1, 128)), full((128, 128)), full((1, 128)),
        ],
        out_specs=[
            pl.BlockSpec((NB, DD), lambda i: (i, 0)),
            pl.BlockSpec((NB, 128), lambda i: (i, 0)),
        ],
        out_shape=[
            jax.ShapeDtypeStruct((NP, DD), jnp.float32),
            jax.ShapeDtypeStruct((NP, 128), jnp.float32),
        ],
    )(h, *zs, x_p, dp, ewp, w2, b2, g2, be2, wu1, bu1, gu, bu, wu2, bu2)


def kernel(x, edge_index, params):
    src = edge_index[0]
    dst = edge_index[1]
    x_p = jnp.pad(x, ((0, NP - NN), (0, 0)))

    ex = params["experts"]
    w1 = jnp.stack([jnp.concatenate([p["W1"][0], p["W1"][1], p["W1"][2]],
                                    axis=0) for p in ex])
    b1 = jnp.stack([p["b1"] for p in ex])[:, None, :]
    g1 = jnp.stack([p["g1"] for p in ex])[:, None, :]
    be1 = jnp.stack([p["be1"] for p in ex])[:, None, :]
    w2 = jnp.stack([jnp.concatenate([p["W2"][0], p["W2"][1]], axis=0)
                    for p in ex])
    b2 = jnp.stack([p["b2"] for p in ex])[:, None, :]
    g2 = jnp.stack([p["g2"] for p in ex])[:, None, :]
    be2 = jnp.stack([p["be2"] for p in ex])[:, None, :]

    wg = params["wg"]
    wg2p = jnp.pad(wg["W2"], ((0, 0), (0, 128 - NE)))
    bg2p = jnp.pad(wg["b2"], (0, 128 - NE), constant_values=-1e30)[None, :]

    ue = params["ue"]
    wu1p = jnp.pad(ue["W1"], ((0, 0), (0, 64)))
    bu1p = jnp.pad(ue["b1"], (0, 64))[None, :]
    gup = jnp.pad(ue["g"], (0, 64))[None, :]
    bup = jnp.pad(ue["b"], (0, 64))[None, :]
    wu2p = jnp.pad(ue["W2"], ((0, 64), (0, 127)))
    bu2p = jnp.pad(ue["b2"], (0, 127))[None, :]

    zeros128 = jnp.zeros((NP, 128), jnp.float32)
    ones40 = jnp.ones((40, 128), jnp.float32)

    dp = _sc_deg(src, zeros128, ones40).reshape(2, NP, 128)
    xs, ewp = _tca(x_p, dp, wg["W1"], wg["b1"][None, :], wg["g"][None, :],
                   wg["b"][None, :], wg2p, bg2p)
    y1 = _sc_prop(xs.reshape(2 * NP, 128), src, dst, zeros128)
    y1 = y1.reshape(2, NP, 128)
    t1s = _tcb(y1, dp)
    y2 = _sc_prop(t1s.reshape(2 * NP, 128), src, dst, zeros128)
    y2 = y2.reshape(2, NP, 128)
    h, hs = _tcc(x_p, y1, y2, dp, w1, b1, g1, be1)
    hs_flat = hs.reshape(6, 2 * NP, 128)
    zs = [_sc_prop(hs_flat[j], src, dst, zeros128).reshape(2, NP, 128)
          for j in range(6)]
    ws, unp = _tcd(h, zs, x_p, dp, ewp, w2, b2, g2, be2,
                   wu1p, bu1p, gup, bup, wu2p, bu2p)
    return ws[:NN], unp[:NN, :1], ewp[:NN, :NE]


# speedup vs baseline: 7.9638x; 2.1535x over previous
import functools

import jax
import jax.numpy as jnp
from jax import lax
from jax.experimental import pallas as pl
from jax.experimental.pallas import tpu as pltpu
from jax.experimental.pallas import tpu_sc as plsc

NN = 10000
NP = 10240
EE = 160000
DD = 256
HH = 512
NE = 3
NB = 1024
GRID = NP // NB

_mesh = plsc.VectorSubcoreMesh(core_axis_name="c", subcore_axis_name="s")


@functools.partial(
    pl.kernel,
    out_type=jax.ShapeDtypeStruct((2 * NP, 128), jnp.float32),
    mesh=_mesh,
    scratch_types=[
        pltpu.VMEM((40,), jnp.int32),
        pltpu.VMEM((40, 128), jnp.float32),
        pltpu.VMEM_SHARED((NP, 128), jnp.float32),
    ],
)
def _sc_deg(src_hbm, zeros_hbm, ones_hbm, out_hbm, idx_v, ones_v, acc):
    c = lax.axis_index("c")
    s = lax.axis_index("s")
    pltpu.sync_copy(zeros_hbm.at[pl.ds(s * 640, 640)],
                    acc.at[pl.ds(s * 640, 640)])
    pltpu.sync_copy(ones_hbm, ones_v)
    plsc.subcore_barrier()
    base = (c * 16 + s) * 5000

    def body(i, _):
        pltpu.sync_copy(src_hbm.at[pl.ds(base + i * 40, 40)], idx_v)
        pltpu.sync_copy(ones_v, acc.at[idx_v], add=True)
        return 0

    lax.fori_loop(0, 125, body, 0)
    plsc.subcore_barrier()
    pltpu.sync_copy(acc.at[pl.ds(s * 640, 640)],
                    out_hbm.at[pl.ds(c * NP + s * 640, 640)])


@functools.partial(
    pl.kernel,
    out_type=jax.ShapeDtypeStruct((2 * NP, 128), jnp.float32),
    mesh=_mesh,
    scratch_types=[
        pltpu.VMEM((10000,), jnp.int32),
        pltpu.VMEM((125, 80), jnp.int32),
        pltpu.VMEM((80, 128), jnp.float32),
        pltpu.VMEM((80, 128), jnp.float32),
        pltpu.VMEM_SHARED((NP, 128), jnp.float32),
        pltpu.SemaphoreType.DMA,
        pltpu.SemaphoreType.DMA,
    ],
)
def _sc_prop(v_hbm, src_hbm, dst2_hbm, zeros_hbm, out_hbm,
             isa, idd, rows0, rows1, acc, sem0, sem1):
    c = lax.axis_index("c")
    s = lax.axis_index("s")
    pltpu.sync_copy(zeros_hbm.at[pl.ds(s * 640, 640)],
                    acc.at[pl.ds(s * 640, 640)])
    pltpu.sync_copy(src_hbm.at[pl.ds(s * 10000, 10000)], isa)
    pltpu.sync_copy(dst2_hbm.at[s], idd)
    off = c * NP

    def adj(j, _):
        isa[pl.ds(j * 16, 16)] = isa[pl.ds(j * 16, 16)] + off
        return 0

    lax.fori_loop(0, 625, adj, 0)
    plsc.subcore_barrier()

    def gat(g, buf, sem):
        pltpu.async_copy(v_hbm.at[isa.at[pl.ds(g * 80, 80)]], buf, sem)

    def wat(buf, sem):
        pltpu.make_async_copy(v_hbm.at[isa.at[pl.ds(0, 80)]], buf,
                              sem).wait()

    def sca(g, buf):
        pltpu.sync_copy(buf, acc.at[idd.at[g]], add=True)

    gat(0, rows0, sem0)

    def body(j, _):
        g0 = 2 * j
        gat(g0 + 1, rows1, sem1)
        wat(rows0, sem0)
        sca(g0, rows0)
        gat(g0 + 2, rows0, sem0)
        wat(rows1, sem1)
        sca(g0 + 1, rows1)
        return 0

    lax.fori_loop(0, 62, body, 0)
    wat(rows0, sem0)
    sca(124, rows0)
    plsc.subcore_barrier()
    pltpu.sync_copy(acc.at[pl.ds(s * 640, 640)],
                    out_hbm.at[pl.ds(c * NP + s * 640, 640)])


def _ln(x, g, b):
    mu = jnp.mean(x, axis=-1, keepdims=True)
    var = jnp.mean((x - mu) ** 2, axis=-1, keepdims=True)
    return (x - mu) / jnp.sqrt(var + 1e-5) * g + b


def _gelu(x):
    return 0.5 * x * (1.0 + lax.erf(x * 0.7071067811865476))


def _dinv(dp_ref):
    deg = dp_ref[0][:, :1] + dp_ref[1][:, :1]
    return jnp.where(deg > 0.0, lax.rsqrt(jnp.where(deg > 0.0, deg, 1.0)),
                     0.0)


def _tca_body(x_ref, dp_ref, wg1_ref, bg1_ref, g_ref, b_ref, wg2_ref,
              bg2_ref, xs_ref, ew_ref):
    dinv = _dinv(dp_ref)
    xs = x_ref[...] * dinv
    xs_ref[0] = xs[:, :128]
    xs_ref[1] = xs[:, 128:]
    w = jnp.dot(x_ref[...], wg1_ref[...],
                preferred_element_type=jnp.float32) + bg1_ref[...]
    w = _gelu(_ln(w, g_ref[...], b_ref[...]))
    logits = jnp.dot(w, wg2_ref[...],
                     preferred_element_type=jnp.float32) + bg2_ref[...]
    m = jnp.max(logits, axis=-1, keepdims=True)
    ex = jnp.exp(logits - m)
    ew_ref[...] = ex / jnp.sum(ex, axis=-1, keepdims=True)


def _tca(x_p, dp, wg1, bg1, g, b, wg2, bg2):
    full = lambda shp: pl.BlockSpec(shp, lambda i: (0,) * len(shp))
    return pl.pallas_call(
        _tca_body,
        grid=(GRID,),
        in_specs=[
            pl.BlockSpec((NB, DD), lambda i: (i, 0)),
            pl.BlockSpec((2, NB, 128), lambda i: (0, i, 0)),
            full((DD, 128)), full((1, 128)), full((1, 128)), full((1, 128)),
            full((128, 128)), full((1, 128)),
        ],
        out_specs=[
            pl.BlockSpec((2, NB, 128), lambda i: (0, i, 0)),
            pl.BlockSpec((NB, 128), lambda i: (i, 0)),
        ],
        out_shape=[
            jax.ShapeDtypeStruct((2, NP, 128), jnp.float32),
            jax.ShapeDtypeStruct((NP, 128), jnp.float32),
        ],
    )(x_p, dp, wg1, bg1, g, b, wg2, bg2)


def _tcb_body(y1_ref, dp_ref, t_ref):
    dinv = _dinv(dp_ref)
    sc = -(dinv * dinv)
    t_ref[0] = y1_ref[0] * sc
    t_ref[1] = y1_ref[1] * sc


def _tcb(y1, dp):
    return pl.pallas_call(
        _tcb_body,
        grid=(GRID,),
        in_specs=[
            pl.BlockSpec((2, NB, 128), lambda i: (0, i, 0)),
            pl.BlockSpec((2, NB, 128), lambda i: (0, i, 0)),
        ],
        out_specs=pl.BlockSpec((2, NB, 128), lambda i: (0, i, 0)),
        out_shape=jax.ShapeDtypeStruct((2, NP, 128), jnp.float32),
    )(y1, dp)


def _tcc_body(x_ref, y1_ref, y2_ref, dp_ref, w1_ref, b1_ref, g1_ref,
              be1_ref, h_ref, hs_ref):
    dinv = _dinv(dp_ref)
    x = x_ref[...]
    tx1 = -dinv * jnp.concatenate([y1_ref[0], y1_ref[1]], axis=1)
    tx2 = -2.0 * dinv * jnp.concatenate([y2_ref[0], y2_ref[1]], axis=1) - x
    cat = jnp.concatenate([x, tx1, tx2], axis=1)
    for e in range(NE):
        pre = jnp.dot(cat, w1_ref[e],
                      preferred_element_type=jnp.float32) + b1_ref[e]
        h = _gelu(_ln(pre, g1_ref[e], be1_ref[e]))
        h_ref[e] = h
        hs = h * dinv
        for j in range(4):
            ci = e * 4 + j
            hs_ref[ci // 2, ci % 2] = hs[:, j * 128:(j + 1) * 128]


def _tcc(x_p, y1, y2, dp, w1, b1, g1, be1):
    full = lambda shp: pl.BlockSpec(shp, lambda i: (0,) * len(shp))
    return pl.pallas_call(
        _tcc_body,
        grid=(GRID,),
        in_specs=[
            pl.BlockSpec((NB, DD), lambda i: (i, 0)),
            pl.BlockSpec((2, NB, 128), lambda i: (0, i, 0)),
            pl.BlockSpec((2, NB, 128), lambda i: (0, i, 0)),
            pl.BlockSpec((2, NB, 128), lambda i: (0, i, 0)),
            full((NE, 3 * DD, HH)), full((NE, 1, HH)),
            full((NE, 1, HH)), full((NE, 1, HH)),
        ],
        out_specs=[
            pl.BlockSpec((NE, NB, HH), lambda i: (0, i, 0)),
            pl.BlockSpec((6, 2, NB, 128), lambda i: (0, 0, i, 0)),
        ],
        out_shape=[
            jax.ShapeDtypeStruct((NE, NP, HH), jnp.float32),
            jax.ShapeDtypeStruct((6, 2, NP, 128), jnp.float32),
        ],
    )(x_p, y1, y2, dp, w1, b1, g1, be1)


def _tcd_body(h_ref, z0, z1, z2, z3, z4, z5, x_ref, dp_ref, ew_ref,
              w2_ref, b2_ref, g2_ref, be2_ref, wu1_ref, bu1_ref, gu_ref,
              bu_ref, wu2_ref, bu2_ref, ws_ref, un_ref):
    dinv = _dinv(dp_ref)
    zrefs = (z0, z1, z2, z3, z4, z5)
    ws = jnp.zeros((NB, DD), jnp.float32)
    ew = ew_ref[...]
    for e in range(NE):
        zcols = [zrefs[(4 * e + j) // 2][(4 * e + j) % 2] for j in range(4)]
        ph = -dinv * jnp.concatenate(zcols, axis=1)
        cat = jnp.concatenate([h_ref[e], ph], axis=1)
        pre = jnp.dot(cat, w2_ref[e],
                      preferred_element_type=jnp.float32) + b2_ref[e]
        o = _ln(pre, g2_ref[e], be2_ref[e])
        ws = ws + o * ew[:, e:e + 1]
    ws_ref[...] = ws
    ucat = jnp.concatenate([ws, x_ref[...]], axis=1)
    u = jnp.dot(ucat, wu1_ref[...],
                preferred_element_type=jnp.float32) + bu1_ref[...]
    lane = lax.broadcasted_iota(jnp.int32, (1, 128), 1)
    m = lane < 64
    mu = jnp.sum(jnp.where(m, u, 0.0), axis=-1, keepdims=True) / 64.0
    d = jnp.where(m, u - mu, 0.0)
    var = jnp.sum(d * d, axis=-1, keepdims=True) / 64.0
    un = (u - mu) / jnp.sqrt(var + 1e-5) * gu_ref[...] + bu_ref[...]
    un = jnp.where(m, _gelu(un), 0.0)
    u2 = jnp.dot(un, wu2_ref[...],
                 preferred_element_type=jnp.float32) + bu2_ref[...]
    un_ref[...] = 1.0 / (1.0 + jnp.exp(-u2))


def _tcd(h, zs, x_p, dp, ewp, w2, b2, g2, be2, wu1, bu1, gu, bu, wu2, bu2):
    full = lambda shp: pl.BlockSpec(shp, lambda i: (0,) * len(shp))
    zspec = pl.BlockSpec((2, NB, 128), lambda i: (0, i, 0))
    return pl.pallas_call(
        _tcd_body,
        grid=(GRID,),
        in_specs=[
            pl.BlockSpec((NE, NB, HH), lambda i: (0, i, 0)),
            zspec, zspec, zspec, zspec, zspec, zspec,
            pl.BlockSpec((NB, DD), lambda i: (i, 0)),
            pl.BlockSpec((2, NB, 128), lambda i: (0, i, 0)),
            pl.BlockSpec((NB, 128), lambda i: (i, 0)),
            full((NE, 2 * HH, DD)), full((NE, 1, DD)),
            full((NE, 1, DD)), full((NE, 1, DD)),
            full((2 * DD, 128)), full((1, 128)), full((1, 128)),
            full((1, 128)), full((128, 128)), full((1, 128)),
        ],
        out_specs=[
            pl.BlockSpec((NB, DD), lambda i: (i, 0)),
            pl.BlockSpec((NB, 128), lambda i: (i, 0)),
        ],
        out_shape=[
            jax.ShapeDtypeStruct((NP, DD), jnp.float32),
            jax.ShapeDtypeStruct((NP, 128), jnp.float32),
        ],
    )(h, *zs, x_p, dp, ewp, w2, b2, g2, be2, wu1, bu1, gu, bu, wu2, bu2)


def kernel(x, edge_index, params):
    src = edge_index[0]
    dst = edge_index[1]
    dst2 = dst.reshape(16, 125, 80)
    x_p = jnp.pad(x, ((0, NP - NN), (0, 0)))

    ex = params["experts"]
    w1 = jnp.stack([jnp.concatenate([p["W1"][0], p["W1"][1], p["W1"][2]],
                                    axis=0) for p in ex])
    b1 = jnp.stack([p["b1"] for p in ex])[:, None, :]
    g1 = jnp.stack([p["g1"] for p in ex])[:, None, :]
    be1 = jnp.stack([p["be1"] for p in ex])[:, None, :]
    w2 = jnp.stack([jnp.concatenate([p["W2"][0], p["W2"][1]], axis=0)
                    for p in ex])
    b2 = jnp.stack([p["b2"] for p in ex])[:, None, :]
    g2 = jnp.stack([p["g2"] for p in ex])[:, None, :]
    be2 = jnp.stack([p["be2"] for p in ex])[:, None, :]

    wg = params["wg"]
    wg2p = jnp.pad(wg["W2"], ((0, 0), (0, 128 - NE)))
    bg2p = jnp.pad(wg["b2"], (0, 128 - NE), constant_values=-1e30)[None, :]

    ue = params["ue"]
    wu1p = jnp.pad(ue["W1"], ((0, 0), (0, 64)))
    bu1p = jnp.pad(ue["b1"], (0, 64))[None, :]
    gup = jnp.pad(ue["g"], (0, 64))[None, :]
    bup = jnp.pad(ue["b"], (0, 64))[None, :]
    wu2p = jnp.pad(ue["W2"], ((0, 64), (0, 127)))
    bu2p = jnp.pad(ue["b2"], (0, 127))[None, :]

    zeros128 = jnp.zeros((NP, 128), jnp.float32)
    ones40 = jnp.ones((40, 128), jnp.float32)

    dp = _sc_deg(src, zeros128, ones40).reshape(2, NP, 128)
    xs, ewp = _tca(x_p, dp, wg["W1"], wg["b1"][None, :], wg["g"][None, :],
                   wg["b"][None, :], wg2p, bg2p)
    y1 = _sc_prop(xs.reshape(2 * NP, 128), src, dst2, zeros128)
    y1 = y1.reshape(2, NP, 128)
    t1s = _tcb(y1, dp)
    y2 = _sc_prop(t1s.reshape(2 * NP, 128), src, dst2, zeros128)
    y2 = y2.reshape(2, NP, 128)
    h, hs = _tcc(x_p, y1, y2, dp, w1, b1, g1, be1)
    hs_flat = hs.reshape(6, 2 * NP, 128)
    zs = [_sc_prop(hs_flat[j], src, dst2, zeros128).reshape(2, NP, 128)
          for j in range(6)]
    ws, unp = _tcd(h, zs, x_p, dp, ewp, w2, b2, g2, be2,
                   wu1p, bu1p, gup, bup, wu2p, bu2p)
    return ws[:NN], unp[:NN, :1], ewp[:NN, :NE]


# speedup vs baseline: 8.5650x; 1.0755x over previous
import functools

import jax
import jax.numpy as jnp
from jax import lax
from jax.experimental import pallas as pl
from jax.experimental.pallas import tpu as pltpu
from jax.experimental.pallas import tpu_sc as plsc

NN = 10000
NP = 10240
EE = 160000
DD = 256
HH = 512
NE = 3
NB = 1024
GRID = NP // NB

_mesh = plsc.VectorSubcoreMesh(core_axis_name="c", subcore_axis_name="s")


@functools.partial(
    pl.kernel,
    out_type=jax.ShapeDtypeStruct((2 * NP, 128), jnp.float32),
    mesh=_mesh,
    scratch_types=[
        pltpu.VMEM((50, 100), jnp.int32),
        pltpu.VMEM((100, 128), jnp.float32),
        pltpu.VMEM_SHARED((NP, 128), jnp.float32),
        pltpu.SemaphoreType.DMA,
        pltpu.SemaphoreType.DMA,
    ],
)
def _sc_deg(src2_hbm, zeros_hbm, ones_hbm, out_hbm, idd, ones_v, acc,
            sem0, sem1):
    c = lax.axis_index("c")
    s = lax.axis_index("s")
    pltpu.sync_copy(zeros_hbm.at[pl.ds(s * 640, 640)],
                    acc.at[pl.ds(s * 640, 640)])
    pltpu.sync_copy(src2_hbm.at[c * 16 + s], idd)
    pltpu.sync_copy(ones_hbm, ones_v)
    plsc.subcore_barrier()

    def ssc(g, sem):
        pltpu.async_copy(ones_v, acc.at[idd.at[g]], sem, add=True)

    def wsc(sem):
        pltpu.make_async_copy(ones_v, acc.at[idd.at[0]], sem).wait()

    ssc(0, sem0)
    ssc(1, sem1)

    def body(j, _):
        wsc(sem0)
        ssc(2 * j + 2, sem0)
        wsc(sem1)
        ssc(2 * j + 3, sem1)
        return 0

    lax.fori_loop(0, 24, body, 0)
    wsc(sem0)
    wsc(sem1)
    plsc.subcore_barrier()
    pltpu.sync_copy(acc.at[pl.ds(s * 640, 640)],
                    out_hbm.at[pl.ds(c * NP + s * 640, 640)])


def _make_prop(npair):
    @functools.partial(
        pl.kernel,
        out_type=jax.ShapeDtypeStruct((2 * npair * NP, 128), jnp.float32),
        mesh=_mesh,
        scratch_types=[
            pltpu.VMEM((10000,), jnp.int32),
            pltpu.VMEM((125, 80), jnp.int32),
            pltpu.VMEM((80, 128), jnp.float32),
            pltpu.VMEM((80, 128), jnp.float32),
            pltpu.VMEM_SHARED((NP, 128), jnp.float32),
            pltpu.SemaphoreType.DMA,
            pltpu.SemaphoreType.DMA,
        ],
    )
    def prop(v_hbm, src_hbm, dst2_hbm, zeros_hbm, out_hbm,
             isa, idd, rows0, rows1, acc, sem0, sem1):
        c = lax.axis_index("c")
        s = lax.axis_index("s")
        pltpu.sync_copy(src_hbm.at[pl.ds(s * 10000, 10000)], isa)
        pltpu.sync_copy(dst2_hbm.at[s], idd)

        def gat(g, buf, sem):
            pltpu.async_copy(v_hbm.at[isa.at[pl.ds(g * 80, 80)]], buf, sem)

        def wat(buf, sem):
            pltpu.make_async_copy(v_hbm.at[isa.at[pl.ds(0, 80)]], buf,
                                  sem).wait()

        def sca(g, buf):
            pltpu.sync_copy(buf, acc.at[idd.at[g]], add=True)

        def pair(pi, _):
            pltpu.sync_copy(zeros_hbm.at[pl.ds(s * 640, 640)],
                            acc.at[pl.ds(s * 640, 640)])
            off = (2 * pi + c) * NP
            dadj = jnp.where(pi == 0, c * NP, 2 * NP)

            def adj(j, _):
                isa[pl.ds(j * 16, 16)] = isa[pl.ds(j * 16, 16)] + dadj
                return 0

            lax.fori_loop(0, 625, adj, 0)
            plsc.subcore_barrier()
            gat(0, rows0, sem0)

            def body(j, _):
                g0 = 2 * j
                gat(g0 + 1, rows1, sem1)
                wat(rows0, sem0)
                sca(g0, rows0)
                gat(g0 + 2, rows0, sem0)
                wat(rows1, sem1)
                sca(g0 + 1, rows1)
                return 0

            lax.fori_loop(0, 62, body, 0)
            wat(rows0, sem0)
            sca(124, rows0)
            plsc.subcore_barrier()
            pltpu.sync_copy(acc.at[pl.ds(s * 640, 640)],
                            out_hbm.at[pl.ds(off + s * 640, 640)])
            return 0

        lax.fori_loop(0, npair, pair, 0)

    return prop


_sc_prop = _make_prop(1)
_sc_prop6 = _make_prop(6)


def _ln(x, g, b):
    mu = jnp.mean(x, axis=-1, keepdims=True)
    var = jnp.mean((x - mu) ** 2, axis=-1, keepdims=True)
    return (x - mu) / jnp.sqrt(var + 1e-5) * g + b


def _gelu(x):
    return 0.5 * x * (1.0 + lax.erf(x * 0.7071067811865476))


def _dinv(dp_ref):
    deg = dp_ref[0][:, :1] + dp_ref[1][:, :1]
    return jnp.where(deg > 0.0, lax.rsqrt(jnp.where(deg > 0.0, deg, 1.0)),
                     0.0)


def _tca_body(x_ref, dp_ref, wg1_ref, bg1_ref, g_ref, b_ref, wg2_ref,
              bg2_ref, xs_ref, ew_ref):
    dinv = _dinv(dp_ref)
    xs = x_ref[...] * dinv
    xs_ref[0] = xs[:, :128]
    xs_ref[1] = xs[:, 128:]
    w = jnp.dot(x_ref[...], wg1_ref[...],
                preferred_element_type=jnp.float32) + bg1_ref[...]
    w = _gelu(_ln(w, g_ref[...], b_ref[...]))
    logits = jnp.dot(w, wg2_ref[...],
                     preferred_element_type=jnp.float32) + bg2_ref[...]
    m = jnp.max(logits, axis=-1, keepdims=True)
    ex = jnp.exp(logits - m)
    ew_ref[...] = ex / jnp.sum(ex, axis=-1, keepdims=True)


def _tca(x_p, dp, wg1, bg1, g, b, wg2, bg2):
    full = lambda shp: pl.BlockSpec(shp, lambda i: (0,) * len(shp))
    return pl.pallas_call(
        _tca_body,
        grid=(GRID,),
        in_specs=[
            pl.BlockSpec((NB, DD), lambda i: (i, 0)),
            pl.BlockSpec((2, NB, 128), lambda i: (0, i, 0)),
            full((DD, 128)), full((1, 128)), full((1, 128)), full((1, 128)),
            full((128, 128)), full((1, 128)),
        ],
        out_specs=[
            pl.BlockSpec((2, NB, 128), lambda i: (0, i, 0)),
            pl.BlockSpec((NB, 128), lambda i: (i, 0)),
        ],
        out_shape=[
            jax.ShapeDtypeStruct((2, NP, 128), jnp.float32),
            jax.ShapeDtypeStruct((NP, 128), jnp.float32),
        ],
    )(x_p, dp, wg1, bg1, g, b, wg2, bg2)


def _tcb_body(y1_ref, dp_ref, t_ref):
    dinv = _dinv(dp_ref)
    sc = -(dinv * dinv)
    t_ref[0] = y1_ref[0] * sc
    t_ref[1] = y1_ref[1] * sc


def _tcb(y1, dp):
    return pl.pallas_call(
        _tcb_body,
        grid=(GRID,),
        in_specs=[
            pl.BlockSpec((2, NB, 128), lambda i: (0, i, 0)),
            pl.BlockSpec((2, NB, 128), lambda i: (0, i, 0)),
        ],
        out_specs=pl.BlockSpec((2, NB, 128), lambda i: (0, i, 0)),
        out_shape=jax.ShapeDtypeStruct((2, NP, 128), jnp.float32),
    )(y1, dp)


def _tcc_body(x_ref, y1_ref, y2_ref, dp_ref, w1_ref, b1_ref, g1_ref,
              be1_ref, h_ref, hs_ref):
    dinv = _dinv(dp_ref)
    x = x_ref[...]
    tx1 = -dinv * jnp.concatenate([y1_ref[0], y1_ref[1]], axis=1)
    tx2 = -2.0 * dinv * jnp.concatenate([y2_ref[0], y2_ref[1]], axis=1) - x
    cat = jnp.concatenate([x, tx1, tx2], axis=1)
    for e in range(NE):
        pre = jnp.dot(cat, w1_ref[e],
                      preferred_element_type=jnp.float32) + b1_ref[e]
        h = _gelu(_ln(pre, g1_ref[e], be1_ref[e]))
        h_ref[e] = h
        hs = h * dinv
        for j in range(4):
            ci = e * 4 + j
            hs_ref[ci // 2, ci % 2] = hs[:, j * 128:(j + 1) * 128]


def _tcc(x_p, y1, y2, dp, w1, b1, g1, be1):
    full = lambda shp: pl.BlockSpec(shp, lambda i: (0,) * len(shp))
    return pl.pallas_call(
        _tcc_body,
        grid=(GRID,),
        in_specs=[
            pl.BlockSpec((NB, DD), lambda i: (i, 0)),
            pl.BlockSpec((2, NB, 128), lambda i: (0, i, 0)),
            pl.BlockSpec((2, NB, 128), lambda i: (0, i, 0)),
            pl.BlockSpec((2, NB, 128), lambda i: (0, i, 0)),
            full((NE, 3 * DD, HH)), full((NE, 1, HH)),
            full((NE, 1, HH)), full((NE, 1, HH)),
        ],
        out_specs=[
            pl.BlockSpec((NE, NB, HH), lambda i: (0, i, 0)),
            pl.BlockSpec((6, 2, NB, 128), lambda i: (0, 0, i, 0)),
        ],
        out_shape=[
            jax.ShapeDtypeStruct((NE, NP, HH), jnp.float32),
            jax.ShapeDtypeStruct((6, 2, NP, 128), jnp.float32),
        ],
    )(x_p, y1, y2, dp, w1, b1, g1, be1)


def _tcd_body(h_ref, z0, z1, z2, z3, z4, z5, x_ref, dp_ref, ew_ref,
              w2_ref, b2_ref, g2_ref, be2_ref, wu1_ref, bu1_ref, gu_ref,
              bu_ref, wu2_ref, bu2_ref, ws_ref, un_ref):
    dinv = _dinv(dp_ref)
    zrefs = (z0, z1, z2, z3, z4, z5)
    ws = jnp.zeros((NB, DD), jnp.float32)
    ew = ew_ref[...]
    for e in range(NE):
        zcols = [zrefs[(4 * e + j) // 2][(4 * e + j) % 2] for j in range(4)]
        ph = -dinv * jnp.concatenate(zcols, axis=1)
        cat = jnp.concatenate([h_ref[e], ph], axis=1)
        pre = jnp.dot(cat, w2_ref[e],
                      preferred_element_type=jnp.float32) + b2_ref[e]
        o = _ln(pre, g2_ref[e], be2_ref[e])
        ws = ws + o * ew[:, e:e + 1]
    ws_ref[...] = ws
    ucat = jnp.concatenate([ws, x_ref[...]], axis=1)
    u = jnp.dot(ucat, wu1_ref[...],
                preferred_element_type=jnp.float32) + bu1_ref[...]
    lane = lax.broadcasted_iota(jnp.int32, (1, 128), 1)
    m = lane < 64
    mu = jnp.sum(jnp.where(m, u, 0.0), axis=-1, keepdims=True) / 64.0
    d = jnp.where(m, u - mu, 0.0)
    var = jnp.sum(d * d, axis=-1, keepdims=True) / 64.0
    un = (u - mu) / jnp.sqrt(var + 1e-5) * gu_ref[...] + bu_ref[...]
    un = jnp.where(m, _gelu(un), 0.0)
    u2 = jnp.dot(un, wu2_ref[...],
                 preferred_element_type=jnp.float32) + bu2_ref[...]
    un_ref[...] = 1.0 / (1.0 + jnp.exp(-u2))


def _tcd(h, zs, x_p, dp, ewp, w2, b2, g2, be2, wu1, bu1, gu, bu, wu2, bu2):
    full = lambda shp: pl.BlockSpec(shp, lambda i: (0,) * len(shp))
    zspec = pl.BlockSpec((2, NB, 128), lambda i: (0, i, 0))
    return pl.pallas_call(
        _tcd_body,
        grid=(GRID,),
        in_specs=[
            pl.BlockSpec((NE, NB, HH), lambda i: (0, i, 0)),
            zspec, zspec, zspec, zspec, zspec, zspec,
            pl.BlockSpec((NB, DD), lambda i: (i, 0)),
            pl.BlockSpec((2, NB, 128), lambda i: (0, i, 0)),
            pl.BlockSpec((NB, 128), lambda i: (i, 0)),
            full((NE, 2 * HH, DD)), full((NE, 1, DD)),
            full((NE, 1, DD)), full((NE, 1, DD)),
            full((2 * DD, 128)), full((1, 128)), full((1, 128)),
            full((1, 128)), full((128, 128)), full((1, 128)),
        ],
        out_specs=[
            pl.BlockSpec((NB, DD), lambda i: (i, 0)),
            pl.BlockSpec((NB, 128), lambda i: (i, 0)),
        ],
        out_shape=[
            jax.ShapeDtypeStruct((NP, DD), jnp.float32),
            jax.ShapeDtypeStruct((NP, 128), jnp.float32),
        ],
    )(h, *zs, x_p, dp, ewp, w2, b2, g2, be2, wu1, bu1, gu, bu, wu2, bu2)


def kernel(x, edge_index, params):
    src = edge_index[0]
    dst = edge_index[1]
    dst2 = dst.reshape(16, 125, 80)
    src2 = src.reshape(32, 50, 100)
    x_p = jnp.pad(x, ((0, NP - NN), (0, 0)))

    ex = params["experts"]
    w1 = jnp.stack([jnp.concatenate([p["W1"][0], p["W1"][1], p["W1"][2]],
                                    axis=0) for p in ex])
    b1 = jnp.stack([p["b1"] for p in ex])[:, None, :]
    g1 = jnp.stack([p["g1"] for p in ex])[:, None, :]
    be1 = jnp.stack([p["be1"] for p in ex])[:, None, :]
    w2 = jnp.stack([jnp.concatenate([p["W2"][0], p["W2"][1]], axis=0)
                    for p in ex])
    b2 = jnp.stack([p["b2"] for p in ex])[:, None, :]
    g2 = jnp.stack([p["g2"] for p in ex])[:, None, :]
    be2 = jnp.stack([p["be2"] for p in ex])[:, None, :]

    wg = params["wg"]
    wg2p = jnp.pad(wg["W2"], ((0, 0), (0, 128 - NE)))
    bg2p = jnp.pad(wg["b2"], (0, 128 - NE), constant_values=-1e30)[None, :]

    ue = params["ue"]
    wu1p = jnp.pad(ue["W1"], ((0, 0), (0, 64)))
    bu1p = jnp.pad(ue["b1"], (0, 64))[None, :]
    gup = jnp.pad(ue["g"], (0, 64))[None, :]
    bup = jnp.pad(ue["b"], (0, 64))[None, :]
    wu2p = jnp.pad(ue["W2"], ((0, 64), (0, 127)))
    bu2p = jnp.pad(ue["b2"], (0, 127))[None, :]

    zeros128 = jnp.zeros((NP, 128), jnp.float32)
    ones100 = jnp.ones((100, 128), jnp.float32)

    dp = _sc_deg(src2, zeros128, ones100).reshape(2, NP, 128)
    xs, ewp = _tca(x_p, dp, wg["W1"], wg["b1"][None, :], wg["g"][None, :],
                   wg["b"][None, :], wg2p, bg2p)
    y1 = _sc_prop(xs.reshape(2 * NP, 128), src, dst2, zeros128)
    y1 = y1.reshape(2, NP, 128)
    t1s = _tcb(y1, dp)
    y2 = _sc_prop(t1s.reshape(2 * NP, 128), src, dst2, zeros128)
    y2 = y2.reshape(2, NP, 128)
    h, hs = _tcc(x_p, y1, y2, dp, w1, b1, g1, be1)
    z12 = _sc_prop6(hs.reshape(12 * NP, 128), src, dst2, zeros128)
    z12 = z12.reshape(6, 2, NP, 128)
    zs = [z12[j] for j in range(6)]
    ws, unp = _tcd(h, zs, x_p, dp, ewp, w2, b2, g2, be2,
                   wu1p, bu1p, gup, bup, wu2p, bu2p)
    return ws[:NN], unp[:NN, :1], ewp[:NN, :NE]
